# baseline (speedup 1.0000x reference)
import functools

import jax
import jax.numpy as jnp
from jax import lax
from jax.experimental import pallas as pl
from jax.experimental.pallas import tpu as pltpu
from jax.experimental.pallas import tpu_sc as plsc

NCLS = 100000
EMD = 256
ALPHA = 0.1
NROW = 16384
BCOL = 1024

NWORK = 32
CPT = NCLS // NWORK
WSZ = 192
NWIN = -(-CPT // WSZ)
WMAGIC = 43691
BATCH = 64
TRASH = WSZ
LANES = 16
SEG = 2048


def _sload(ref, i):
  return ref[pl.ds(i, LANES)][0]


def _sc_update_body(rows_hbm, tgt_hbm, nc_ref,
                    tgtc, own, win, table, counts, rowbuf, oldbuf,
                    idxb, posb, clsb, invb, pres_pos, pres_cnt, sem):
  cid = lax.axis_index("c")
  sid = lax.axis_index("s")
  wid = sid * 2 + cid
  lo = wid * CPT

  lane = lax.iota(jnp.int32, LANES)
  zeros_i = jnp.zeros((LANES,), jnp.int32)
  zeros_f = jnp.zeros((LANES,), jnp.float32)
  one_row = jnp.where(lane == 0, 1.0, 0.0).astype(jnp.float32)

  def _zi(i, c):
    win[pl.ds(i * LANES, LANES)] = zeros_i
    return c
  lax.fori_loop(0, (NROW + LANES) // LANES, _zi, 0)

  def _zt(i, c):
    for ch in range(EMD // LANES):
      table[i, pl.ds(ch * LANES, LANES)] = zeros_f
    counts[pl.ds(i * LANES, LANES)] = zeros_f
    return c
  lax.fori_loop(0, WSZ + 1, _zt, 0)

  def seg_body(seg, cur):
    pltpu.sync_copy(tgt_hbm.at[pl.ds(seg * SEG, SEG)], tgtc)

    def scan_body(ch, cur):
      t = tgtc[pl.ds(ch * LANES, LANES)]
      rel = t - lo
      m = (rel >= 0) & (rel < CPT)
      rid = seg * SEG + ch * LANES + lane
      v = (rel * 16384) + rid
      mi = m.astype(jnp.int32)
      pos = plsc.cumsum(mi) - 1
      dest = jnp.where(m, cur + pos, NROW)
      plsc.store_scatter(own, [dest], v)
      return cur + jnp.sum(mi)
    return lax.fori_loop(0, SEG // LANES, scan_body, cur)
  own_cnt = lax.fori_loop(0, NROW // SEG, seg_body, 0)
  own_chunks = (own_cnt + (LANES - 1)) >> 4

  def window_body(w, _carry):
    def wscan_body(ch, cur):
      vo = own[pl.ds(ch * LANES, LANES)]
      valid = (ch * LANES + lane) < own_cnt
      wv = ((vo >> 14) * WMAGIC) >> 23
      m = (wv == w) & valid
      mi = m.astype(jnp.int32)
      pos = plsc.cumsum(mi) - 1
      dest = jnp.where(m, cur + pos, NROW)
      plsc.store_scatter(win, [dest], vo)
      return cur + jnp.sum(mi)
    k = lax.fori_loop(0, own_chunks, wscan_body, 0)

    nb = (k + (BATCH - 1)) >> 6
    def batch_body(b, _c):
      b0 = b * BATCH
      for ch in range(BATCH // LANES):
        vw = win[pl.ds(b0 + ch * LANES, LANES)]
        valid = (b0 + ch * LANES + lane) < k
        relw = (vw >> 14) - w * WSZ
        idxb[pl.ds(ch * LANES, LANES)] = vw & (16384 - 1)
        posb[pl.ds(ch * LANES, LANES)] = jnp.where(valid, relw, TRASH)
      pltpu.async_copy(rows_hbm.at[idxb], rowbuf, sem).wait()

      def acc_body(r, _r):
        pr = _sload(posb, r)
        for ch in range(EMD // LANES):
          tv = table[pr, pl.ds(ch * LANES, LANES)]
          rv = rowbuf[r, pl.ds(ch * LANES, LANES)]
          table[pr, pl.ds(ch * LANES, LANES)] = tv + rv
        counts[pl.ds(pr * LANES, LANES)] = (
            counts[pl.ds(pr * LANES, LANES)] + one_row)
        return _r
      lax.fori_loop(0, BATCH, acc_body, 0)
      return _c
    lax.fori_loop(0, nb, batch_body, 0)

    base = lo + w * WSZ
    def pres_body(chv, p):
      c_idx = chv * LANES + lane
      cnt = plsc.load_gather(counts, [c_idx * LANES])
      m = cnt > 0.0
      mi = m.astype(jnp.int32)
      pos = plsc.cumsum(mi) - 1
      dest = jnp.where(m, p + pos, WSZ + LANES)
      plsc.store_scatter(pres_pos, [dest], c_idx)
      plsc.store_scatter(pres_cnt, [dest], cnt)
      return p + jnp.sum(mi)
    p = lax.fori_loop(0, WSZ // LANES, pres_body, 0)

    last = jnp.maximum(p - 1, 0)
    last_pos = _sload(pres_pos, last)
    last_cnt = _sload(pres_cnt, last)

    nb2 = (p + (BATCH - 1)) >> 6
    def blend_body(b, _c):
      b0 = b * BATCH
      for ch in range(BATCH // LANES):
        off = b0 + ch * LANES
        valid = (off + lane) < p
        pos16 = jnp.where(valid, pres_pos[pl.ds(off, LANES)], last_pos)
        cnt16 = jnp.where(valid, pres_cnt[pl.ds(off, LANES)], last_cnt)
        posb[pl.ds(ch * LANES, LANES)] = pos16
        clsb[pl.ds(ch * LANES, LANES)] = base + pos16
        invb[pl.ds(ch * LANES, LANES)] = ALPHA / cnt16
      pltpu.async_copy(nc_ref.at[clsb], oldbuf, sem).wait()

      def row_body(r, _r):
        pr = _sload(posb, r)
        iv = _sload(invb, r)
        for ch in range(EMD // LANES):
          old = oldbuf[r, pl.ds(ch * LANES, LANES)]
          tv = table[pr, pl.ds(ch * LANES, LANES)]
          oldbuf[r, pl.ds(ch * LANES, LANES)] = old * (1.0 - ALPHA) + tv * iv
        return _r
      lax.fori_loop(0, BATCH, row_body, 0)

      pltpu.sync_copy(oldbuf, nc_ref.at[clsb])

      def zero_body(r, _r):
        pr = _sload(posb, r)
        for ch in range(EMD // LANES):
          table[pr, pl.ds(ch * LANES, LANES)] = zeros_f
        counts[pl.ds(pr * LANES, LANES)] = zeros_f
        return _r
      lax.fori_loop(0, jnp.minimum(BATCH, p - b0), zero_body, 0)
      return _c
    lax.fori_loop(0, nb2, blend_body, 0)

    for ch in range(EMD // LANES):
      table[TRASH, pl.ds(ch * LANES, LANES)] = zeros_f
    counts[pl.ds(TRASH * LANES, LANES)] = zeros_f
    return _carry

  lax.fori_loop(0, NWIN, window_body, 0)


def _make_sc_update():
  mesh = plsc.VectorSubcoreMesh(core_axis_name="c", subcore_axis_name="s")
  return pl.kernel(
      _sc_update_body,
      out_type=(),
      mesh=mesh,
      scratch_types=[
          pltpu.VMEM((SEG,), jnp.int32),
          pltpu.VMEM((NROW + LANES,), jnp.int32),
          pltpu.VMEM((NROW + LANES,), jnp.int32),
          pltpu.VMEM((WSZ + 1, EMD), jnp.float32),
          pltpu.VMEM(((WSZ + 1) * LANES,), jnp.float32),
          pltpu.VMEM((BATCH, EMD), jnp.float32),
          pltpu.VMEM((BATCH, EMD), jnp.float32),
          pltpu.VMEM((BATCH,), jnp.int32),
          pltpu.VMEM((BATCH + LANES,), jnp.int32),
          pltpu.VMEM((BATCH,), jnp.int32),
          pltpu.VMEM((BATCH + LANES,), jnp.float32),
          pltpu.VMEM((WSZ + 2 * LANES,), jnp.int32),
          pltpu.VMEM((WSZ + 2 * LANES,), jnp.float32),
          pltpu.SemaphoreType.DMA,
      ],
      compiler_params=pltpu.CompilerParams(needs_layout_passes=False),
      name="sc_center_update",
  )


def _dt_gather_body(tgt_hbm, nc_hbm, g_out, idxg, gbuf, sem):
  cid = lax.axis_index("c")
  sid = lax.axis_index("s")
  wid = sid * 2 + cid
  off = wid * (BCOL // NWORK)
  pltpu.sync_copy(tgt_hbm.at[pl.ds(off, BCOL // NWORK)], idxg)
  pltpu.async_copy(nc_hbm.at[idxg], gbuf, sem).wait()
  pltpu.sync_copy(gbuf, g_out.at[pl.ds(off, BCOL // NWORK)])


def _make_dt_gather():
  mesh = plsc.VectorSubcoreMesh(core_axis_name="c", subcore_axis_name="s")
  return pl.kernel(
      _dt_gather_body,
      out_type=jax.ShapeDtypeStruct((BCOL, EMD), jnp.float32),
      mesh=mesh,
      scratch_types=[
          pltpu.VMEM((BCOL // NWORK,), jnp.int32),
          pltpu.VMEM((BCOL // NWORK, EMD), jnp.float32),
          pltpu.SemaphoreType.DMA,
      ],
      compiler_params=pltpu.CompilerParams(needs_layout_passes=False),
      name="dt_gather",
  )


TILE = 1024
NTILE = -(-NCLS // TILE)


def _colsum_body(nc_ref, xcol_ref, colsum_ref, xn_s):
  j = pl.program_id(0)

  @pl.when(j == 0)
  def _():
    x = xcol_ref[...]
    nrm = jnp.sqrt(jnp.sum(x * x, axis=1, keepdims=True))
    xn_s[...] = x / jnp.maximum(nrm, 1e-12)
    colsum_ref[...] = jnp.zeros_like(colsum_ref)

  rows = j * TILE + lax.broadcasted_iota(jnp.int32, (TILE, 1), 0)
  maskr = rows < NCLS
  c = jnp.where(maskr, nc_ref[...], 0.0)
  nrm = jnp.sqrt(jnp.sum(c * c, axis=1, keepdims=True))
  cn = c / jnp.maximum(nrm, 1e-12)
  dist = lax.dot_general(cn, xn_s[...], (((1,), (1,)), ((), ())),
                         preferred_element_type=jnp.float32)
  e = jnp.where(maskr, jnp.exp(dist), 0.0)
  colsum_ref[...] += jnp.sum(e, axis=0, keepdims=True)


def _make_colsum():
  return pl.pallas_call(
      _colsum_body,
      grid=(NTILE,),
      in_specs=[
          pl.BlockSpec((TILE, EMD), lambda j: (j, 0)),
          pl.BlockSpec((BCOL, EMD), lambda j: (0, 0)),
      ],
      out_specs=pl.BlockSpec((1, BCOL), lambda j: (0, 0)),
      out_shape=jax.ShapeDtypeStruct((1, BCOL), jnp.float32),
      scratch_shapes=[pltpu.VMEM((BCOL, EMD), jnp.float32)],
      compiler_params=pltpu.CompilerParams(
          dimension_semantics=("arbitrary",)),
      name="colsum_dist",
  )


def _smooth_body(ep_ref, colsum_ref, g_ref, xcol_ref, out_ref):
  ep = ep_ref[0]
  x = xcol_ref[...]
  xn = x / jnp.maximum(jnp.sqrt(jnp.sum(x * x, axis=1, keepdims=True)), 1e-12)
  g = g_ref[...]
  gn = g / jnp.maximum(jnp.sqrt(jnp.sum(g * g, axis=1, keepdims=True)), 1e-12)
  dgx = lax.dot_general(gn, xn, (((1,), (1,)), ((), ())),
                        preferred_element_type=jnp.float32)
  rows = lax.broadcasted_iota(jnp.int32, (BCOL, 1), 0)
  cols = lax.broadcasted_iota(jnp.int32, (BCOL, BCOL), 1)
  dt = jnp.sum(jnp.where(rows == cols, dgx, 0.0), axis=0, keepdims=True)
  denom = jnp.float32(NCLS) + colsum_ref[...]
  s = (1.0 + jnp.exp(dt)) / denom
  sm = jnp.minimum(jnp.exp(s * ep.astype(jnp.float32)) / NCLS, 0.1)
  out_ref[...] = jnp.where(ep == 0, 0.0, sm)


def _make_smooth():
  return pl.pallas_call(
      _smooth_body,
      in_specs=[
          pl.BlockSpec(memory_space=pltpu.SMEM),
          pl.BlockSpec((1, BCOL), lambda: (0, 0)),
          pl.BlockSpec((BCOL, EMD), lambda: (0, 0)),
          pl.BlockSpec((BCOL, EMD), lambda: (0, 0)),
      ],
      out_specs=pl.BlockSpec((1, BCOL), lambda: (0, 0)),
      out_shape=jax.ShapeDtypeStruct((1, BCOL), jnp.float32),
      name="smooth_rate",
  )


_sc_update = _make_sc_update()
_dt_gather = _make_dt_gather()
_colsum = _make_colsum()
_smooth = _make_smooth()


def kernel(inputs_col, targets_col, inputs_row, target_row, epoch, center):
  tgt_row = target_row.astype(jnp.int32)
  tgt_col = targets_col.astype(jnp.int32)

  nc_ref = jax.new_ref(center)
  _sc_update(inputs_row, tgt_row, nc_ref)
  new_center = nc_ref[...]

  colsum = _colsum(new_center, inputs_col)
  g = _dt_gather(tgt_col, new_center)
  ep = jnp.asarray(epoch, jnp.int32).reshape(1)
  smooth = _smooth(ep, colsum, g, inputs_col).reshape(BCOL)
  return (new_center, smooth)


# speedup vs baseline: 1.5775x; 1.5775x over previous
import functools

import jax
import jax.numpy as jnp
from jax import lax
from jax.experimental import pallas as pl
from jax.experimental.pallas import tpu as pltpu
from jax.experimental.pallas import tpu_sc as plsc

NCLS = 100000
EMD = 256
ALPHA = 0.1
NROW = 16384
BCOL = 1024

NWORK = 32
CPT = NCLS // NWORK
WSZ = 192
NWIN = -(-CPT // WSZ)
WMAGIC = 43691
BATCH = 32
TRASH = WSZ
LANES = 16
SEG = 2048


def _sload(ref, i):
  return ref[pl.ds(i, LANES)][0]


def _sc_update_body(rows_hbm, tgt_hbm, nc_ref,
                    tgtc, own, win, table, counts, rowbuf, oldbuf,
                    idxb, posb, clsb, invb, pres_pos, pres_cnt, sem):
  cid = lax.axis_index("c")
  sid = lax.axis_index("s")
  wid = sid * 2 + cid
  lo = wid * CPT

  lane = lax.iota(jnp.int32, LANES)
  zeros_i = jnp.zeros((LANES,), jnp.int32)
  zeros_f = jnp.zeros((LANES,), jnp.float32)
  one_row = jnp.where(lane == 0, 1.0, 0.0).astype(jnp.float32)

  def _zi(i, c):
    win[pl.ds(i * LANES, LANES)] = zeros_i
    return c
  lax.fori_loop(0, (NROW + LANES) // LANES, _zi, 0)

  def _zt(i, c):
    for ch in range(EMD // LANES):
      table[i, pl.ds(ch * LANES, LANES)] = zeros_f
    counts[pl.ds(i * LANES, LANES)] = zeros_f
    return c
  lax.fori_loop(0, WSZ + 1, _zt, 0)

  def seg_body(seg, cur):
    pltpu.sync_copy(tgt_hbm.at[pl.ds(seg * SEG, SEG)], tgtc)

    def scan_body(ch, cur):
      t = tgtc[pl.ds(ch * LANES, LANES)]
      rel = t - lo
      m = (rel >= 0) & (rel < CPT)
      rid = seg * SEG + ch * LANES + lane
      v = (rel * 16384) + rid
      mi = m.astype(jnp.int32)
      pos = plsc.cumsum(mi) - 1
      dest = jnp.where(m, cur + pos, NROW)
      plsc.store_scatter(own, [dest], v)
      return cur + jnp.sum(mi)
    return lax.fori_loop(0, SEG // LANES, scan_body, cur)
  own_cnt = lax.fori_loop(0, NROW // SEG, seg_body, 0)
  own_chunks = (own_cnt + (LANES - 1)) >> 4

  def window_body(w, _carry):
    def wscan_body(ch, cur):
      vo = own[pl.ds(ch * LANES, LANES)]
      valid = (ch * LANES + lane) < own_cnt
      wv = ((vo >> 14) * WMAGIC) >> 23
      m = (wv == w) & valid
      mi = m.astype(jnp.int32)
      pos = plsc.cumsum(mi) - 1
      dest = jnp.where(m, cur + pos, NROW)
      plsc.store_scatter(win, [dest], vo)
      return cur + jnp.sum(mi)
    k = lax.fori_loop(0, own_chunks, wscan_body, 0)

    nb = (k + (BATCH - 1)) >> 5
    def batch_body(b, _c):
      b0 = b * BATCH
      for ch in range(BATCH // LANES):
        vw = win[pl.ds(b0 + ch * LANES, LANES)]
        valid = (b0 + ch * LANES + lane) < k
        relw = (vw >> 14) - w * WSZ
        idxb[pl.ds(ch * LANES, LANES)] = vw & (16384 - 1)
        posb[pl.ds(ch * LANES, LANES)] = jnp.where(valid, relw, TRASH)
      pltpu.async_copy(rows_hbm.at[idxb], rowbuf, sem).wait()

      def acc_body(r, _r):
        pr = _sload(posb, r)
        for ch in range(EMD // LANES):
          tv = table[pr, pl.ds(ch * LANES, LANES)]
          rv = rowbuf[r, pl.ds(ch * LANES, LANES)]
          table[pr, pl.ds(ch * LANES, LANES)] = tv + rv
        counts[pl.ds(pr * LANES, LANES)] = (
            counts[pl.ds(pr * LANES, LANES)] + one_row)
        return _r
      lax.fori_loop(0, BATCH, acc_body, 0)
      return _c
    lax.fori_loop(0, nb, batch_body, 0)

    base = lo + w * WSZ
    def pres_body(chv, p):
      c_idx = chv * LANES + lane
      cnt = plsc.load_gather(counts, [c_idx * LANES])
      m = cnt > 0.0
      mi = m.astype(jnp.int32)
      pos = plsc.cumsum(mi) - 1
      dest = jnp.where(m, p + pos, WSZ + LANES)
      plsc.store_scatter(pres_pos, [dest], c_idx)
      plsc.store_scatter(pres_cnt, [dest], cnt)
      return p + jnp.sum(mi)
    p = lax.fori_loop(0, WSZ // LANES, pres_body, 0)

    last = jnp.maximum(p - 1, 0)
    last_pos = _sload(pres_pos, last)
    last_cnt = _sload(pres_cnt, last)

    nb2 = (p + (BATCH - 1)) >> 5
    def blend_body(b, _c):
      b0 = b * BATCH
      for ch in range(BATCH // LANES):
        off = b0 + ch * LANES
        valid = (off + lane) < p
        pos16 = jnp.where(valid, pres_pos[pl.ds(off, LANES)], last_pos)
        cnt16 = jnp.where(valid, pres_cnt[pl.ds(off, LANES)], last_cnt)
        posb[pl.ds(ch * LANES, LANES)] = pos16
        clsb[pl.ds(ch * LANES, LANES)] = base + pos16
        invb[pl.ds(ch * LANES, LANES)] = ALPHA / cnt16
      pltpu.async_copy(nc_ref.at[clsb], oldbuf, sem).wait()

      def row_body(r, _r):
        pr = _sload(posb, r)
        iv = _sload(invb, r)
        for ch in range(EMD // LANES):
          old = oldbuf[r, pl.ds(ch * LANES, LANES)]
          tv = table[pr, pl.ds(ch * LANES, LANES)]
          oldbuf[r, pl.ds(ch * LANES, LANES)] = old * (1.0 - ALPHA) + tv * iv
        return _r
      lax.fori_loop(0, BATCH, row_body, 0)

      pltpu.sync_copy(oldbuf, nc_ref.at[clsb])

      def zero_body(r, _r):
        pr = _sload(posb, r)
        for ch in range(EMD // LANES):
          table[pr, pl.ds(ch * LANES, LANES)] = zeros_f
        counts[pl.ds(pr * LANES, LANES)] = zeros_f
        return _r
      lax.fori_loop(0, jnp.minimum(BATCH, p - b0), zero_body, 0)
      return _c
    lax.fori_loop(0, nb2, blend_body, 0)

    for ch in range(EMD // LANES):
      table[TRASH, pl.ds(ch * LANES, LANES)] = zeros_f
    counts[pl.ds(TRASH * LANES, LANES)] = zeros_f
    return _carry

  lax.fori_loop(0, NWIN, window_body, 0)


def _make_sc_update():
  mesh = plsc.VectorSubcoreMesh(core_axis_name="c", subcore_axis_name="s")
  return pl.kernel(
      _sc_update_body,
      out_type=(),
      mesh=mesh,
      scratch_types=[
          pltpu.VMEM((SEG,), jnp.int32),
          pltpu.VMEM((NROW + LANES,), jnp.int32),
          pltpu.VMEM((NROW + LANES,), jnp.int32),
          pltpu.VMEM((WSZ + 1, EMD), jnp.float32),
          pltpu.VMEM(((WSZ + 1) * LANES,), jnp.float32),
          pltpu.VMEM((BATCH, EMD), jnp.float32),
          pltpu.VMEM((BATCH, EMD), jnp.float32),
          pltpu.VMEM((BATCH,), jnp.int32),
          pltpu.VMEM((BATCH + LANES,), jnp.int32),
          pltpu.VMEM((BATCH,), jnp.int32),
          pltpu.VMEM((BATCH + LANES,), jnp.float32),
          pltpu.VMEM((WSZ + 2 * LANES,), jnp.int32),
          pltpu.VMEM((WSZ + 2 * LANES,), jnp.float32),
          pltpu.SemaphoreType.DMA,
      ],
      compiler_params=pltpu.CompilerParams(needs_layout_passes=False),
      name="sc_center_update",
  )


def _dt_gather_body(tgt_hbm, nc_hbm, g_out, idxg, gbuf, sem):
  cid = lax.axis_index("c")
  sid = lax.axis_index("s")
  wid = sid * 2 + cid
  off = wid * (BCOL // NWORK)
  pltpu.sync_copy(tgt_hbm.at[pl.ds(off, BCOL // NWORK)], idxg)
  pltpu.async_copy(nc_hbm.at[idxg], gbuf, sem).wait()
  pltpu.sync_copy(gbuf, g_out.at[pl.ds(off, BCOL // NWORK)])


def _make_dt_gather():
  mesh = plsc.VectorSubcoreMesh(core_axis_name="c", subcore_axis_name="s")
  return pl.kernel(
      _dt_gather_body,
      out_type=jax.ShapeDtypeStruct((BCOL, EMD), jnp.float32),
      mesh=mesh,
      scratch_types=[
          pltpu.VMEM((BCOL // NWORK,), jnp.int32),
          pltpu.VMEM((BCOL // NWORK, EMD), jnp.float32),
          pltpu.SemaphoreType.DMA,
      ],
      compiler_params=pltpu.CompilerParams(needs_layout_passes=False),
      name="dt_gather",
  )


TILE = 1024
NTILE = -(-NCLS // TILE)


def _colsum_body(nc_ref, xcol_ref, colsum_ref, xn_s):
  j = pl.program_id(0)

  @pl.when(j == 0)
  def _():
    x = xcol_ref[...]
    nrm = jnp.sqrt(jnp.sum(x * x, axis=1, keepdims=True))
    xn_s[...] = x / jnp.maximum(nrm, 1e-12)
    colsum_ref[...] = jnp.zeros_like(colsum_ref)

  rows = j * TILE + lax.broadcasted_iota(jnp.int32, (TILE, 1), 0)
  maskr = rows < NCLS
  c = jnp.where(maskr, nc_ref[...], 0.0)
  nrm = jnp.sqrt(jnp.sum(c * c, axis=1, keepdims=True))
  cn = c / jnp.maximum(nrm, 1e-12)
  dist = lax.dot_general(cn, xn_s[...], (((1,), (1,)), ((), ())),
                         preferred_element_type=jnp.float32)
  e = jnp.where(maskr, jnp.exp(dist), 0.0)
  colsum_ref[...] += jnp.sum(e, axis=0, keepdims=True)


def _make_colsum():
  return pl.pallas_call(
      _colsum_body,
      grid=(NTILE,),
      in_specs=[
          pl.BlockSpec((TILE, EMD), lambda j: (j, 0)),
          pl.BlockSpec((BCOL, EMD), lambda j: (0, 0)),
      ],
      out_specs=pl.BlockSpec((1, BCOL), lambda j: (0, 0)),
      out_shape=jax.ShapeDtypeStruct((1, BCOL), jnp.float32),
      scratch_shapes=[pltpu.VMEM((BCOL, EMD), jnp.float32)],
      compiler_params=pltpu.CompilerParams(
          dimension_semantics=("arbitrary",)),
      name="colsum_dist",
  )


def _smooth_body(ep_ref, colsum_ref, g_ref, xcol_ref, out_ref):
  ep = ep_ref[0]
  x = xcol_ref[...]
  xn = x / jnp.maximum(jnp.sqrt(jnp.sum(x * x, axis=1, keepdims=True)), 1e-12)
  g = g_ref[...]
  gn = g / jnp.maximum(jnp.sqrt(jnp.sum(g * g, axis=1, keepdims=True)), 1e-12)
  dgx = lax.dot_general(gn, xn, (((1,), (1,)), ((), ())),
                        preferred_element_type=jnp.float32)
  rows = lax.broadcasted_iota(jnp.int32, (BCOL, 1), 0)
  cols = lax.broadcasted_iota(jnp.int32, (BCOL, BCOL), 1)
  dt = jnp.sum(jnp.where(rows == cols, dgx, 0.0), axis=0, keepdims=True)
  denom = jnp.float32(NCLS) + colsum_ref[...]
  s = (1.0 + jnp.exp(dt)) / denom
  sm = jnp.minimum(jnp.exp(s * ep.astype(jnp.float32)) / NCLS, 0.1)
  out_ref[...] = jnp.where(ep == 0, 0.0, sm)


def _make_smooth():
  return pl.pallas_call(
      _smooth_body,
      in_specs=[
          pl.BlockSpec(memory_space=pltpu.SMEM),
          pl.BlockSpec((1, BCOL), lambda: (0, 0)),
          pl.BlockSpec((BCOL, EMD), lambda: (0, 0)),
          pl.BlockSpec((BCOL, EMD), lambda: (0, 0)),
      ],
      out_specs=pl.BlockSpec((1, BCOL), lambda: (0, 0)),
      out_shape=jax.ShapeDtypeStruct((1, BCOL), jnp.float32),
      name="smooth_rate",
  )


_sc_update = _make_sc_update()
_dt_gather = _make_dt_gather()
_colsum = _make_colsum()
_smooth = _make_smooth()


def kernel(inputs_col, targets_col, inputs_row, target_row, epoch, center):
  tgt_row = target_row.astype(jnp.int32)
  tgt_col = targets_col.astype(jnp.int32)

  nc_ref = jax.new_ref(center)
  _sc_update(inputs_row, tgt_row, nc_ref)
  new_center = nc_ref[...]

  colsum = _colsum(new_center, inputs_col)
  g = _dt_gather(tgt_col, new_center)
  ep = jnp.asarray(epoch, jnp.int32).reshape(1)
  smooth = _smooth(ep, colsum, g, inputs_col).reshape(BCOL)
  return (new_center, smooth)


# speedup vs baseline: 1.5858x; 1.0053x over previous
import functools

import jax
import jax.numpy as jnp
from jax import lax
from jax.experimental import pallas as pl
from jax.experimental.pallas import tpu as pltpu
from jax.experimental.pallas import tpu_sc as plsc

NCLS = 100000
EMD = 256
ALPHA = 0.1
NROW = 16384
BCOL = 1024

NWORK = 32
CPT = NCLS // NWORK
WSZ = 192
NWIN = -(-CPT // WSZ)
WMAGIC = 43691
BATCH = 32
TRASH = WSZ
LANES = 16
SEG = 2048


def _sload(ref, i):
  return ref[pl.ds(i, LANES)][0]


def _sc_update_body(rows_hbm, tgt_hbm, nc_ref,
                    tgtc, own, win, table, counts, rowbuf, oldbuf,
                    idxb, posb, clsb, invb, pres_pos, pres_cnt, sem, sem2):
  cid = lax.axis_index("c")
  sid = lax.axis_index("s")
  wid = sid * 2 + cid
  lo = wid * CPT

  lane = lax.iota(jnp.int32, LANES)
  zeros_i = jnp.zeros((LANES,), jnp.int32)
  zeros_f = jnp.zeros((LANES,), jnp.float32)
  one_row = jnp.where(lane == 0, 1.0, 0.0).astype(jnp.float32)

  def _zi(i, c):
    win[pl.ds(i * LANES, LANES)] = zeros_i
    return c
  lax.fori_loop(0, (NROW + LANES) // LANES, _zi, 0)

  def _zt(i, c):
    for ch in range(EMD // LANES):
      table[i, pl.ds(ch * LANES, LANES)] = zeros_f
    counts[pl.ds(i * LANES, LANES)] = zeros_f
    return c
  lax.fori_loop(0, WSZ + 1, _zt, 0)

  def seg_body(seg, cur):
    pltpu.sync_copy(tgt_hbm.at[pl.ds(seg * SEG, SEG)], tgtc)

    def scan_body(ch, cur):
      t = tgtc[pl.ds(ch * LANES, LANES)]
      rel = t - lo
      m = (rel >= 0) & (rel < CPT)
      rid = seg * SEG + ch * LANES + lane
      v = (rel * 16384) + rid
      mi = m.astype(jnp.int32)
      pos = plsc.cumsum(mi) - 1
      dest = jnp.where(m, cur + pos, NROW)
      plsc.store_scatter(own, [dest], v)
      return cur + jnp.sum(mi)
    return lax.fori_loop(0, SEG // LANES, scan_body, cur)
  own_cnt = lax.fori_loop(0, NROW // SEG, seg_body, 0)
  own_chunks = (own_cnt + (LANES - 1)) >> 4

  def window_body(w, pend):
    def wscan_body(ch, cur):
      vo = own[pl.ds(ch * LANES, LANES)]
      valid = (ch * LANES + lane) < own_cnt
      wv = ((vo >> 14) * WMAGIC) >> 23
      m = (wv == w) & valid
      mi = m.astype(jnp.int32)
      pos = plsc.cumsum(mi) - 1
      dest = jnp.where(m, cur + pos, NROW)
      plsc.store_scatter(win, [dest], vo)
      return cur + jnp.sum(mi)
    k = lax.fori_loop(0, own_chunks, wscan_body, 0)

    nb = (k + (BATCH - 1)) >> 5
    def batch_body(b, _c):
      b0 = b * BATCH
      for ch in range(BATCH // LANES):
        vw = win[pl.ds(b0 + ch * LANES, LANES)]
        valid = (b0 + ch * LANES + lane) < k
        relw = (vw >> 14) - w * WSZ
        idxb[pl.ds(ch * LANES, LANES)] = vw & (16384 - 1)
        posb[pl.ds(ch * LANES, LANES)] = jnp.where(valid, relw, TRASH)
      pltpu.async_copy(rows_hbm.at[idxb], rowbuf, sem).wait()

      def acc_body(r, _r):
        pr = _sload(posb, r)
        for ch in range(EMD // LANES):
          tv = table[pr, pl.ds(ch * LANES, LANES)]
          rv = rowbuf[r, pl.ds(ch * LANES, LANES)]
          table[pr, pl.ds(ch * LANES, LANES)] = tv + rv
        counts[pl.ds(pr * LANES, LANES)] = (
            counts[pl.ds(pr * LANES, LANES)] + one_row)
        return _r
      lax.fori_loop(0, BATCH, acc_body, 0)
      return _c
    lax.fori_loop(0, nb, batch_body, 0)

    base = lo + w * WSZ
    def pres_body(chv, p):
      c_idx = chv * LANES + lane
      cnt = plsc.load_gather(counts, [c_idx * LANES])
      m = cnt > 0.0
      mi = m.astype(jnp.int32)
      pos = plsc.cumsum(mi) - 1
      dest = jnp.where(m, p + pos, WSZ + LANES)
      plsc.store_scatter(pres_pos, [dest], c_idx)
      plsc.store_scatter(pres_cnt, [dest], cnt)
      return p + jnp.sum(mi)
    p = lax.fori_loop(0, WSZ // LANES, pres_body, 0)

    last = jnp.maximum(p - 1, 0)
    last_pos = _sload(pres_pos, last)
    last_cnt = _sload(pres_cnt, last)

    nb2 = (p + (BATCH - 1)) >> 5
    def blend_body(b, pend):
      b0 = b * BATCH
      @pl.when(pend > 0)
      def _():
        pltpu.make_async_copy(oldbuf, nc_ref.at[clsb], sem2).wait()
      for ch in range(BATCH // LANES):
        off = b0 + ch * LANES
        valid = (off + lane) < p
        pos16 = jnp.where(valid, pres_pos[pl.ds(off, LANES)], last_pos)
        cnt16 = jnp.where(valid, pres_cnt[pl.ds(off, LANES)], last_cnt)
        posb[pl.ds(ch * LANES, LANES)] = pos16
        clsb[pl.ds(ch * LANES, LANES)] = base + pos16
        invb[pl.ds(ch * LANES, LANES)] = ALPHA / cnt16
      pltpu.async_copy(nc_ref.at[clsb], oldbuf, sem).wait()

      def row_body(r, _r):
        pr = _sload(posb, r)
        iv = _sload(invb, r)
        for ch in range(EMD // LANES):
          old = oldbuf[r, pl.ds(ch * LANES, LANES)]
          tv = table[pr, pl.ds(ch * LANES, LANES)]
          oldbuf[r, pl.ds(ch * LANES, LANES)] = old * (1.0 - ALPHA) + tv * iv
        return _r
      lax.fori_loop(0, BATCH, row_body, 0)

      def zero_body(r, _r):
        pr = _sload(posb, r)
        for ch in range(EMD // LANES):
          table[pr, pl.ds(ch * LANES, LANES)] = zeros_f
        counts[pl.ds(pr * LANES, LANES)] = zeros_f
        return _r
      lax.fori_loop(0, jnp.minimum(BATCH, p - b0), zero_body, 0)

      pltpu.async_copy(oldbuf, nc_ref.at[clsb], sem2)
      return 1
    pend2 = lax.fori_loop(0, nb2, blend_body, pend)

    for ch in range(EMD // LANES):
      table[TRASH, pl.ds(ch * LANES, LANES)] = zeros_f
    counts[pl.ds(TRASH * LANES, LANES)] = zeros_f
    return pend2

  pend_fin = lax.fori_loop(0, NWIN, window_body, 0)

  @pl.when(pend_fin > 0)
  def _():
    pltpu.make_async_copy(oldbuf, nc_ref.at[clsb], sem2).wait()


def _make_sc_update():
  mesh = plsc.VectorSubcoreMesh(core_axis_name="c", subcore_axis_name="s")
  return pl.kernel(
      _sc_update_body,
      out_type=(),
      mesh=mesh,
      scratch_types=[
          pltpu.VMEM((SEG,), jnp.int32),
          pltpu.VMEM((NROW + LANES,), jnp.int32),
          pltpu.VMEM((NROW + LANES,), jnp.int32),
          pltpu.VMEM((WSZ + 1, EMD), jnp.float32),
          pltpu.VMEM(((WSZ + 1) * LANES,), jnp.float32),
          pltpu.VMEM((BATCH, EMD), jnp.float32),
          pltpu.VMEM((BATCH, EMD), jnp.float32),
          pltpu.VMEM((BATCH,), jnp.int32),
          pltpu.VMEM((BATCH + LANES,), jnp.int32),
          pltpu.VMEM((BATCH,), jnp.int32),
          pltpu.VMEM((BATCH + LANES,), jnp.float32),
          pltpu.VMEM((WSZ + 2 * LANES,), jnp.int32),
          pltpu.VMEM((WSZ + 2 * LANES,), jnp.float32),
          pltpu.SemaphoreType.DMA,
          pltpu.SemaphoreType.DMA,
      ],
      compiler_params=pltpu.CompilerParams(needs_layout_passes=False),
      name="sc_center_update",
  )


def _dt_gather_body(tgt_hbm, nc_hbm, g_out, idxg, gbuf, sem):
  cid = lax.axis_index("c")
  sid = lax.axis_index("s")
  wid = sid * 2 + cid
  off = wid * (BCOL // NWORK)
  pltpu.sync_copy(tgt_hbm.at[pl.ds(off, BCOL // NWORK)], idxg)
  pltpu.async_copy(nc_hbm.at[idxg], gbuf, sem).wait()
  pltpu.sync_copy(gbuf, g_out.at[pl.ds(off, BCOL // NWORK)])


def _make_dt_gather():
  mesh = plsc.VectorSubcoreMesh(core_axis_name="c", subcore_axis_name="s")
  return pl.kernel(
      _dt_gather_body,
      out_type=jax.ShapeDtypeStruct((BCOL, EMD), jnp.float32),
      mesh=mesh,
      scratch_types=[
          pltpu.VMEM((BCOL // NWORK,), jnp.int32),
          pltpu.VMEM((BCOL // NWORK, EMD), jnp.float32),
          pltpu.SemaphoreType.DMA,
      ],
      compiler_params=pltpu.CompilerParams(needs_layout_passes=False),
      name="dt_gather",
  )


TILE = 1024
NTILE = -(-NCLS // TILE)


def _colsum_body(nc_ref, xcol_ref, colsum_ref, xn_s):
  j = pl.program_id(0)

  @pl.when(j == 0)
  def _():
    x = xcol_ref[...]
    nrm = jnp.sqrt(jnp.sum(x * x, axis=1, keepdims=True))
    xn_s[...] = x / jnp.maximum(nrm, 1e-12)
    colsum_ref[...] = jnp.zeros_like(colsum_ref)

  rows = j * TILE + lax.broadcasted_iota(jnp.int32, (TILE, 1), 0)
  maskr = rows < NCLS
  c = jnp.where(maskr, nc_ref[...], 0.0)
  nrm = jnp.sqrt(jnp.sum(c * c, axis=1, keepdims=True))
  cn = c / jnp.maximum(nrm, 1e-12)
  dist = lax.dot_general(cn, xn_s[...], (((1,), (1,)), ((), ())),
                         preferred_element_type=jnp.float32)
  e = jnp.where(maskr, jnp.exp(dist), 0.0)
  colsum_ref[...] += jnp.sum(e, axis=0, keepdims=True)


def _make_colsum():
  return pl.pallas_call(
      _colsum_body,
      grid=(NTILE,),
      in_specs=[
          pl.BlockSpec((TILE, EMD), lambda j: (j, 0)),
          pl.BlockSpec((BCOL, EMD), lambda j: (0, 0)),
      ],
      out_specs=pl.BlockSpec((1, BCOL), lambda j: (0, 0)),
      out_shape=jax.ShapeDtypeStruct((1, BCOL), jnp.float32),
      scratch_shapes=[pltpu.VMEM((BCOL, EMD), jnp.float32)],
      compiler_params=pltpu.CompilerParams(
          dimension_semantics=("arbitrary",)),
      name="colsum_dist",
  )


def _smooth_body(ep_ref, colsum_ref, g_ref, xcol_ref, out_ref):
  ep = ep_ref[0]
  x = xcol_ref[...]
  xn = x / jnp.maximum(jnp.sqrt(jnp.sum(x * x, axis=1, keepdims=True)), 1e-12)
  g = g_ref[...]
  gn = g / jnp.maximum(jnp.sqrt(jnp.sum(g * g, axis=1, keepdims=True)), 1e-12)
  dgx = lax.dot_general(gn, xn, (((1,), (1,)), ((), ())),
                        preferred_element_type=jnp.float32)
  rows = lax.broadcasted_iota(jnp.int32, (BCOL, 1), 0)
  cols = lax.broadcasted_iota(jnp.int32, (BCOL, BCOL), 1)
  dt = jnp.sum(jnp.where(rows == cols, dgx, 0.0), axis=0, keepdims=True)
  denom = jnp.float32(NCLS) + colsum_ref[...]
  s = (1.0 + jnp.exp(dt)) / denom
  sm = jnp.minimum(jnp.exp(s * ep.astype(jnp.float32)) / NCLS, 0.1)
  out_ref[...] = jnp.where(ep == 0, 0.0, sm)


def _make_smooth():
  return pl.pallas_call(
      _smooth_body,
      in_specs=[
          pl.BlockSpec(memory_space=pltpu.SMEM),
          pl.BlockSpec((1, BCOL), lambda: (0, 0)),
          pl.BlockSpec((BCOL, EMD), lambda: (0, 0)),
          pl.BlockSpec((BCOL, EMD), lambda: (0, 0)),
      ],
      out_specs=pl.BlockSpec((1, BCOL), lambda: (0, 0)),
      out_shape=jax.ShapeDtypeStruct((1, BCOL), jnp.float32),
      name="smooth_rate",
  )


_sc_update = _make_sc_update()
_dt_gather = _make_dt_gather()
_colsum = _make_colsum()
_smooth = _make_smooth()


def kernel(inputs_col, targets_col, inputs_row, target_row, epoch, center):
  tgt_row = target_row.astype(jnp.int32)
  tgt_col = targets_col.astype(jnp.int32)

  nc_ref = jax.new_ref(center)
  _sc_update(inputs_row, tgt_row, nc_ref)
  new_center = nc_ref[...]

  colsum = _colsum(new_center, inputs_col)
  g = _dt_gather(tgt_col, new_center)
  ep = jnp.asarray(epoch, jnp.int32).reshape(1)
  smooth = _smooth(ep, colsum, g, inputs_col).reshape(BCOL)
  return (new_center, smooth)


# speedup vs baseline: 1.5945x; 1.0054x over previous
import jax
import jax.numpy as jnp
from jax import lax
from jax.experimental import pallas as pl
from jax.experimental.pallas import tpu as pltpu
from jax.experimental.pallas import tpu_sc as plsc

NCLS = 100000
EMD = 256
ALPHA = 0.1
NROW = 16384
BCOL = 1024

NWORK = 32
CPT = NCLS // NWORK
WSZ = 192
NWIN = -(-CPT // WSZ)
WMAGIC = 43691
BATCH = 32
TRASH = WSZ
LANES = 16
SEG = 2048


def _sload(ref, i):
  return ref[pl.ds(i, LANES)][0]


def _sc_update_body(rows_hbm, tgt_hbm, nc_ref,
                    tgtc, own, win, table, counts, rowbuf, oldbuf,
                    idxb, posb, clsb, invb, pres_pos, pres_cnt, sem, sem2):
  cid = lax.axis_index("c")
  sid = lax.axis_index("s")
  wid = sid * 2 + cid
  lo = wid * CPT

  lane = lax.iota(jnp.int32, LANES)
  zeros_i = jnp.zeros((LANES,), jnp.int32)
  zeros_f = jnp.zeros((LANES,), jnp.float32)
  one_row = jnp.where(lane == 0, 1.0, 0.0).astype(jnp.float32)

  def _zi(i, c):
    win[pl.ds(i * LANES, LANES)] = zeros_i
    return c
  lax.fori_loop(0, (NROW + LANES) // LANES, _zi, 0)

  def _zt(i, c):
    for ch in range(EMD // LANES):
      table[i, pl.ds(ch * LANES, LANES)] = zeros_f
    counts[pl.ds(i * LANES, LANES)] = zeros_f
    return c
  lax.fori_loop(0, WSZ + 1, _zt, 0)

  def seg_body(seg, cur):
    pltpu.sync_copy(tgt_hbm.at[pl.ds(seg * SEG, SEG)], tgtc)

    def scan_body(ch, cur):
      t = tgtc[pl.ds(ch * LANES, LANES)]
      rel = t - lo
      m = (rel >= 0) & (rel < CPT)
      rid = seg * SEG + ch * LANES + lane
      v = (rel * 16384) + rid
      mi = m.astype(jnp.int32)
      pos = plsc.cumsum(mi) - 1
      dest = jnp.where(m, cur + pos, NROW)
      plsc.store_scatter(own, [dest], v)
      return cur + jnp.sum(mi)
    return lax.fori_loop(0, SEG // LANES, scan_body, cur)
  own_cnt = lax.fori_loop(0, NROW // SEG, seg_body, 0)
  own_chunks = (own_cnt + (LANES - 1)) >> 4

  def window_body(w, pend):
    def wscan_body(ch, cur):
      vo = own[pl.ds(ch * LANES, LANES)]
      valid = (ch * LANES + lane) < own_cnt
      wv = ((vo >> 14) * WMAGIC) >> 23
      m = (wv == w) & valid
      mi = m.astype(jnp.int32)
      pos = plsc.cumsum(mi) - 1
      dest = jnp.where(m, cur + pos, NROW)
      plsc.store_scatter(win, [dest], vo)
      return cur + jnp.sum(mi)
    k = lax.fori_loop(0, own_chunks, wscan_body, 0)

    nb = (k + (BATCH - 1)) >> 5
    def batch_body(b, _c):
      b0 = b * BATCH
      for ch in range(BATCH // LANES):
        vw = win[pl.ds(b0 + ch * LANES, LANES)]
        valid = (b0 + ch * LANES + lane) < k
        relw = (vw >> 14) - w * WSZ
        idxb[pl.ds(ch * LANES, LANES)] = vw & (16384 - 1)
        posb[pl.ds(ch * LANES, LANES)] = jnp.where(valid, relw, TRASH)
      pltpu.async_copy(rows_hbm.at[idxb], rowbuf, sem).wait()

      def acc_body(r, _r):
        pr = _sload(posb, r)
        for ch in range(EMD // LANES):
          tv = table[pr, pl.ds(ch * LANES, LANES)]
          rv = rowbuf[r, pl.ds(ch * LANES, LANES)]
          table[pr, pl.ds(ch * LANES, LANES)] = tv + rv
        counts[pl.ds(pr * LANES, LANES)] = (
            counts[pl.ds(pr * LANES, LANES)] + one_row)
        return _r
      lax.fori_loop(0, BATCH, acc_body, 0)
      return _c
    lax.fori_loop(0, nb, batch_body, 0)

    base = lo + w * WSZ
    def pres_body(chv, p):
      c_idx = chv * LANES + lane
      cnt = plsc.load_gather(counts, [c_idx * LANES])
      m = cnt > 0.0
      mi = m.astype(jnp.int32)
      pos = plsc.cumsum(mi) - 1
      dest = jnp.where(m, p + pos, WSZ + LANES)
      plsc.store_scatter(pres_pos, [dest], c_idx)
      plsc.store_scatter(pres_cnt, [dest], cnt)
      return p + jnp.sum(mi)
    p = lax.fori_loop(0, WSZ // LANES, pres_body, 0)

    last = jnp.maximum(p - 1, 0)
    last_pos = _sload(pres_pos, last)
    last_cnt = _sload(pres_cnt, last)

    nb2 = (p + (BATCH - 1)) >> 5
    def blend_body(b, pend):
      b0 = b * BATCH
      @pl.when(pend > 0)
      def _():
        pltpu.make_async_copy(oldbuf, nc_ref.at[clsb], sem2).wait()
      for ch in range(BATCH // LANES):
        off = b0 + ch * LANES
        valid = (off + lane) < p
        pos16 = jnp.where(valid, pres_pos[pl.ds(off, LANES)], last_pos)
        cnt16 = jnp.where(valid, pres_cnt[pl.ds(off, LANES)], last_cnt)
        posb[pl.ds(ch * LANES, LANES)] = pos16
        clsb[pl.ds(ch * LANES, LANES)] = base + pos16
        invb[pl.ds(ch * LANES, LANES)] = ALPHA / cnt16
      pltpu.async_copy(nc_ref.at[clsb], oldbuf, sem).wait()

      def row_body(r, _r):
        pr = _sload(posb, r)
        iv = _sload(invb, r)
        for ch in range(EMD // LANES):
          old = oldbuf[r, pl.ds(ch * LANES, LANES)]
          tv = table[pr, pl.ds(ch * LANES, LANES)]
          oldbuf[r, pl.ds(ch * LANES, LANES)] = old * (1.0 - ALPHA) + tv * iv
        return _r
      lax.fori_loop(0, BATCH, row_body, 0)

      def zero_body(r, _r):
        pr = _sload(posb, r)
        for ch in range(EMD // LANES):
          table[pr, pl.ds(ch * LANES, LANES)] = zeros_f
        counts[pl.ds(pr * LANES, LANES)] = zeros_f
        return _r
      lax.fori_loop(0, jnp.minimum(BATCH, p - b0), zero_body, 0)

      pltpu.async_copy(oldbuf, nc_ref.at[clsb], sem2)
      return 1
    pend2 = lax.fori_loop(0, nb2, blend_body, pend)

    for ch in range(EMD // LANES):
      table[TRASH, pl.ds(ch * LANES, LANES)] = zeros_f
    counts[pl.ds(TRASH * LANES, LANES)] = zeros_f
    return pend2

  pend_fin = lax.fori_loop(0, NWIN, window_body, 0)

  @pl.when(pend_fin > 0)
  def _():
    pltpu.make_async_copy(oldbuf, nc_ref.at[clsb], sem2).wait()


def _make_sc_update():
  mesh = plsc.VectorSubcoreMesh(core_axis_name="c", subcore_axis_name="s")
  return pl.kernel(
      _sc_update_body,
      out_type=(),
      mesh=mesh,
      scratch_types=[
          pltpu.VMEM((SEG,), jnp.int32),
          pltpu.VMEM((NROW + LANES,), jnp.int32),
          pltpu.VMEM((NROW + LANES,), jnp.int32),
          pltpu.VMEM((WSZ + 1, EMD), jnp.float32),
          pltpu.VMEM(((WSZ + 1) * LANES,), jnp.float32),
          pltpu.VMEM((BATCH, EMD), jnp.float32),
          pltpu.VMEM((BATCH, EMD), jnp.float32),
          pltpu.VMEM((BATCH,), jnp.int32),
          pltpu.VMEM((BATCH + LANES,), jnp.int32),
          pltpu.VMEM((BATCH,), jnp.int32),
          pltpu.VMEM((BATCH + LANES,), jnp.float32),
          pltpu.VMEM((WSZ + 2 * LANES,), jnp.int32),
          pltpu.VMEM((WSZ + 2 * LANES,), jnp.float32),
          pltpu.SemaphoreType.DMA,
          pltpu.SemaphoreType.DMA,
      ],
      compiler_params=pltpu.CompilerParams(needs_layout_passes=False),
      name="sc_center_update",
  )


def _dt_gather_body(tgt_hbm, nc_hbm, g_out, idxg, gbuf, sem):
  cid = lax.axis_index("c")
  sid = lax.axis_index("s")
  wid = sid * 2 + cid
  off = wid * (BCOL // NWORK)
  pltpu.sync_copy(tgt_hbm.at[pl.ds(off, BCOL // NWORK)], idxg)
  pltpu.async_copy(nc_hbm.at[idxg], gbuf, sem).wait()
  pltpu.sync_copy(gbuf, g_out.at[pl.ds(off, BCOL // NWORK)])


def _make_dt_gather():
  mesh = plsc.VectorSubcoreMesh(core_axis_name="c", subcore_axis_name="s")
  return pl.kernel(
      _dt_gather_body,
      out_type=jax.ShapeDtypeStruct((BCOL, EMD), jnp.float32),
      mesh=mesh,
      scratch_types=[
          pltpu.VMEM((BCOL // NWORK,), jnp.int32),
          pltpu.VMEM((BCOL // NWORK, EMD), jnp.float32),
          pltpu.SemaphoreType.DMA,
      ],
      compiler_params=pltpu.CompilerParams(needs_layout_passes=False),
      name="dt_gather",
  )


TILE = 1024
NTILE = -(-NCLS // TILE)


def _colsum_body(nc_ref, xcol_ref, colsum_ref, xn_s):
  j = pl.program_id(0)

  @pl.when(j == 0)
  def _():
    x = xcol_ref[...]
    nrm = jnp.sqrt(jnp.sum(x * x, axis=1, keepdims=True))
    xn_s[...] = x / jnp.maximum(nrm, 1e-12)
    colsum_ref[...] = jnp.zeros_like(colsum_ref)

  rows = j * TILE + lax.broadcasted_iota(jnp.int32, (TILE, 1), 0)
  maskr = rows < NCLS
  c = jnp.where(maskr, nc_ref[...], 0.0)
  nrm = jnp.sqrt(jnp.sum(c * c, axis=1, keepdims=True))
  cn = c / jnp.maximum(nrm, 1e-12)
  dist = lax.dot_general(cn, xn_s[...], (((1,), (1,)), ((), ())),
                         preferred_element_type=jnp.float32)
  e = jnp.where(maskr, jnp.exp(dist), 0.0)
  colsum_ref[...] += jnp.sum(e, axis=0, keepdims=True)


def _make_colsum():
  return pl.pallas_call(
      _colsum_body,
      grid=(NTILE,),
      in_specs=[
          pl.BlockSpec((TILE, EMD), lambda j: (j, 0)),
          pl.BlockSpec((BCOL, EMD), lambda j: (0, 0)),
      ],
      out_specs=pl.BlockSpec((1, BCOL), lambda j: (0, 0)),
      out_shape=jax.ShapeDtypeStruct((1, BCOL), jnp.float32),
      scratch_shapes=[pltpu.VMEM((BCOL, EMD), jnp.float32)],
      compiler_params=pltpu.CompilerParams(
          dimension_semantics=("arbitrary",)),
      name="colsum_dist",
  )


def _smooth_body(ep_ref, colsum_ref, g_ref, xcol_ref, out_ref):
  ep = ep_ref[0]
  x = xcol_ref[...]
  xn = x / jnp.maximum(jnp.sqrt(jnp.sum(x * x, axis=1, keepdims=True)), 1e-12)
  g = g_ref[...]
  gn = g / jnp.maximum(jnp.sqrt(jnp.sum(g * g, axis=1, keepdims=True)), 1e-12)
  dgx = lax.dot_general(gn, xn, (((1,), (1,)), ((), ())),
                        preferred_element_type=jnp.float32)
  rows = lax.broadcasted_iota(jnp.int32, (BCOL, 1), 0)
  cols = lax.broadcasted_iota(jnp.int32, (BCOL, BCOL), 1)
  dt = jnp.sum(jnp.where(rows == cols, dgx, 0.0), axis=0, keepdims=True)
  denom = jnp.float32(NCLS) + colsum_ref[...]
  s = (1.0 + jnp.exp(dt)) / denom
  sm = jnp.minimum(jnp.exp(s * ep.astype(jnp.float32)) / NCLS, 0.1)
  out_ref[...] = jnp.where(ep == 0, 0.0, sm)


def _make_smooth():
  return pl.pallas_call(
      _smooth_body,
      in_specs=[
          pl.BlockSpec(memory_space=pltpu.SMEM),
          pl.BlockSpec((1, BCOL), lambda: (0, 0)),
          pl.BlockSpec((BCOL, EMD), lambda: (0, 0)),
          pl.BlockSpec((BCOL, EMD), lambda: (0, 0)),
      ],
      out_specs=pl.BlockSpec((1, BCOL), lambda: (0, 0)),
      out_shape=jax.ShapeDtypeStruct((1, BCOL), jnp.float32),
      name="smooth_rate",
  )


_sc_update = _make_sc_update()
_dt_gather = _make_dt_gather()
_colsum = _make_colsum()
_smooth = _make_smooth()


def kernel(inputs_col, targets_col, inputs_row, target_row, epoch, center):
  tgt_row = target_row.astype(jnp.int32)
  tgt_col = targets_col.astype(jnp.int32)

  nc_ref = jax.new_ref(center)
  _sc_update(inputs_row, tgt_row, nc_ref)
  new_center = nc_ref[...]

  colsum = _colsum(new_center, inputs_col)
  g = _dt_gather(tgt_col, new_center)
  ep = jnp.asarray(epoch, jnp.int32).reshape(1)
  smooth = _smooth(ep, colsum, g, inputs_col).reshape(BCOL)
  return (new_center, smooth)
